# Initial kernel scaffold; baseline (speedup 1.0000x reference)
import jax
import jax.numpy as jnp
from jax.experimental import pallas as pl


def kernel(x, edge_index, edge_attr, pieces, edge_select, golden_edge, props, W_in, b_in, W_msg, W_edge, W_upd, b_upd, W_prop, b_prop, W_add, b_add, W_m1, b_m1, W_m2, b_m2, W_m3, b_m3, W_out, b_out):
    raise NotImplementedError("write your pallas kernel here")


# speedup vs baseline: 6.9768x; 6.9768x over previous
import functools

import jax
import jax.numpy as jnp
from jax import lax
from jax.experimental import pallas as pl
from jax.experimental.pallas import tpu as pltpu
from jax.experimental.pallas import tpu_sc as plsc

B = 32
NODE = 64
NODE_DIM = 160
HID = 64
NET = 5
N = B * NODE
E = 32768
NPAIR = NODE * NODE


_NW = 32
_ROWS = 32
_SLAB = _ROWS * N
_CHUNK = 4096
_NCHUNK = E // _CHUNK


def _sc_body(src_hbm, dst_hbm, ea0, ea1, ea2, ea3, ea4, zro_hbm,
             a_hbm, s_hbm, abuf, sbuf, srcb, dstb, eb0, eb1, eb2, eb3, eb4):
    wid = lax.axis_index("s") * 2 + lax.axis_index("c")

    ones16 = jnp.ones((16,), jnp.float32)
    ea_hbm = (ea0, ea1, ea2, ea3, ea4)
    eabs = (eb0, eb1, eb2, eb3, eb4)

    for p in range(2):
        lo = (wid + p * _NW) * _ROWS
        pltpu.sync_copy(zro_hbm, abuf)
        pltpu.sync_copy(zro_hbm.at[pl.ds(0, _ROWS * 8)], sbuf)

        for chunk in range(_NCHUNK):
            base = chunk * _CHUNK
            pltpu.sync_copy(src_hbm.at[pl.ds(base, _CHUNK)], srcb)
            pltpu.sync_copy(dst_hbm.at[pl.ds(base, _CHUNK)], dstb)
            for k in range(NET):
                pltpu.sync_copy(ea_hbm[k].at[pl.ds(base, _CHUNK)], eabs[k])

            def scan_vec(v, _):
                s = srcb[pl.ds(v * 16, 16)]
                d = dstb[pl.ds(v * 16, 16)]
                dl = d - lo
                okd = (dl >= 0) & (dl < _ROWS)
                dli = jnp.where(okd, dl, 0)
                plsc.addupdate_scatter(
                    abuf, [dli * N + s], ones16, mask=okd)
                soff = dli * 8
                for k in range(NET):
                    vals = eabs[k][pl.ds(v * 16, 16)]
                    plsc.addupdate_scatter(
                        sbuf, [soff + k], vals, mask=okd)
                return _

            lax.fori_loop(0, _CHUNK // 16, scan_vec, None)

        pltpu.sync_copy(abuf, a_hbm.at[pl.ds(lo * N, _SLAB)])
        pltpu.sync_copy(sbuf, s_hbm.at[pl.ds(lo * 8, _ROWS * 8)])


def _build_adjacency(src, dst, ea_cols):
    mesh = plsc.VectorSubcoreMesh(core_axis_name="c", subcore_axis_name="s")
    f = pl.kernel(
        _sc_body,
        out_type=(
            jax.ShapeDtypeStruct((N * N,), jnp.float32),
            jax.ShapeDtypeStruct((N * 8,), jnp.float32),
        ),
        mesh=mesh,
        compiler_params=pltpu.CompilerParams(
            needs_layout_passes=False, use_tc_tiling_on_sc=False),
        scratch_types=[
            pltpu.VMEM((_SLAB,), jnp.float32),
            pltpu.VMEM((_ROWS * 8,), jnp.float32),
            pltpu.VMEM((_CHUNK,), jnp.int32),
            pltpu.VMEM((_CHUNK,), jnp.int32),
            pltpu.VMEM((_CHUNK,), jnp.float32),
            pltpu.VMEM((_CHUNK,), jnp.float32),
            pltpu.VMEM((_CHUNK,), jnp.float32),
            pltpu.VMEM((_CHUNK,), jnp.float32),
            pltpu.VMEM((_CHUNK,), jnp.float32),
        ],
    )
    zro = jnp.zeros((_SLAB,), jnp.float32)
    a, s = f(src, dst, ea_cols[0], ea_cols[1], ea_cols[2], ea_cols[3],
             ea_cols[4], zro)
    return a.reshape(N, N), s.reshape(N, 8)


def _dot(a, b):
    return jnp.dot(a, b, precision=lax.Precision.HIGHEST,
                   preferred_element_type=jnp.float32)


def _tc_body(a_ref, s8_ref, x_ref, props_ref, wproppad_ref, bprop_ref,
             wadd_ref, badd_ref, win_ref, bin_ref, wmsg_ref, wedge8_ref,
             wupd_ref, bupd_ref, wm1_ref, bm1_ref, wm2_ref, bm2_ref,
             wm3_ref, bm3_ref, wout8_ref, bout8_ref, gold_ref,
             out_ref, p_scr, q_scr, h_scr):
    pid = pl.program_id(0)

    @pl.when(pid == 0)
    def _encoder():
        h_scr[...] = jnp.maximum(
            _dot(x_ref[...], win_ref[...]) + bin_ref[...], 0.0)
        c = _dot(s8_ref[...], wedge8_ref[...])
        u1 = wupd_ref[0:HID, :]
        u2 = wupd_ref[HID:2 * HID, :]
        wmsg = wmsg_ref[...]
        bupd = bupd_ref[...]
        for _ in range(4):
            def kstep(kb, acc):
                return acc + _dot(a_ref[:, pl.ds(kb * 128, 128)],
                                  h_scr[pl.ds(kb * 128, 128), :])
            g = lax.fori_loop(0, 16, kstep, jnp.zeros((N, HID), jnp.float32))
            agg = _dot(g, wmsg) + c
            h = h_scr[...]
            h_scr[...] = jnp.maximum(_dot(h, u1) + _dot(agg, u2) + bupd, 0.0)
        prop = _dot(props_ref[...], wproppad_ref[...]) + bprop_ref[...]
        pvec = _dot(prop, wadd_ref[HID:, :]) + badd_ref[...]
        propp = jnp.broadcast_to(
            pvec[:, None, :], (B, NODE, HID)).reshape(N, HID)
        neg = _dot(h_scr[...], wadd_ref[0:HID, :]) + propp
        p_scr[...] = _dot(neg, wm1_ref[0:HID, :]) + bm1_ref[...]
        q_scr[...] = _dot(neg, wm1_ref[HID:, :])
        out_ref[...] = jnp.zeros((1, 1), jnp.float32)

    pb = p_scr[pl.ds(pid * NODE, NODE), :]
    qb = q_scr[pl.ds(pid * NODE, NODE), :]
    h1 = jnp.maximum(
        pb[:, None, :] + qb[None, :, :], 0.0).reshape(NPAIR, HID)
    h2 = jnp.maximum(_dot(h1, wm2_ref[...]) + bm2_ref[...], 0.0)
    w35 = _dot(wm3_ref[...], wout8_ref[...])
    b35 = _dot(bm3_ref[...], wout8_ref[...]) + bout8_ref[...]
    logits = _dot(h2, w35) + b35
    m = jnp.max(logits, axis=1, keepdims=True)
    lse = jnp.log(jnp.sum(jnp.exp(logits - m), axis=1, keepdims=True)) + m
    gold = gold_ref[...].reshape(1, NPAIR)
    onehot_t = (lax.broadcasted_iota(jnp.int32, (8, NPAIR), 0)
                == gold).astype(jnp.float32)
    pickmat = _dot(onehot_t, logits)
    eye8 = (lax.broadcasted_iota(jnp.int32, (8, 8), 0)
            == lax.broadcasted_iota(jnp.int32, (8, 8), 1)).astype(jnp.float32)
    pick = jnp.sum(pickmat * eye8)
    out_ref[...] += jnp.reshape(jnp.sum(lse) - pick, (1, 1))


def _dense_forward(amat, s8, x2d, props_pad, wprop_pad, bprop_row, wadd,
                   badd_row, win, bin_row, wmsg, wedge8, wupd, bupd_row,
                   wm1, bm1_row, wm2, bm2_row, wm3, bm3_row, wout8,
                   bout8_row, gold3, interpret=False):
    full = lambda shape: pl.BlockSpec(shape, lambda b: (0,) * len(shape))
    out = pl.pallas_call(
        _tc_body,
        grid=(B,),
        in_specs=[
            full((N, N)), full((N, 8)), full((N, NODE_DIM)),
            full((B, 128)), full((128, 32)), full((1, 32)),
            full((96, HID)), full((1, HID)),
            full((NODE_DIM, HID)), full((1, HID)),
            full((HID, HID)), full((8, HID)),
            full((2 * HID, HID)), full((1, HID)),
            full((2 * HID, HID)), full((1, HID)),
            full((HID, HID)), full((1, HID)),
            full((HID, 2 * HID)), full((1, 2 * HID)),
            full((2 * HID, 8)), full((1, 8)),
            pl.BlockSpec((1, 1, NPAIR), lambda b: (b, 0, 0)),
        ],
        out_specs=pl.BlockSpec((1, 1), lambda b: (0, 0)),
        out_shape=jax.ShapeDtypeStruct((1, 1), jnp.float32),
        scratch_shapes=[
            pltpu.VMEM((N, HID), jnp.float32),
            pltpu.VMEM((N, HID), jnp.float32),
            pltpu.VMEM((N, HID), jnp.float32),
        ],
        interpret=interpret,
    )(amat, s8, x2d, props_pad, wprop_pad, bprop_row, wadd, badd_row,
      win, bin_row, wmsg, wedge8, wupd, bupd_row, wm1, bm1_row, wm2,
      bm2_row, wm3, bm3_row, wout8, bout8_row, gold3)
    return out


@jax.jit
def kernel(x, edge_index, edge_attr, pieces, edge_select, golden_edge,
           props, W_in, b_in, W_msg, W_edge, W_upd, b_upd, W_prop, b_prop,
           W_add, b_add, W_m1, b_m1, W_m2, b_m2, W_m3, b_m3, W_out, b_out):
    src = edge_index[0].astype(jnp.int32)
    dst = edge_index[1].astype(jnp.int32)
    ea_cols = [edge_attr[:, k] for k in range(NET)]

    amat, s8 = _build_adjacency(src, dst, ea_cols)

    x2d = x.reshape(N, NODE_DIM)
    props_pad = jnp.zeros((B, 128), jnp.float32).at[:, 0:1].set(props)
    wprop_pad = jnp.zeros((128, 32), jnp.float32).at[0:1, :].set(W_prop)
    wedge8 = jnp.zeros((8, HID), jnp.float32).at[0:NET, :].set(W_edge)
    wout8 = jnp.zeros((2 * HID, 8), jnp.float32).at[:, 0:NET].set(W_out)
    bout8 = jnp.full((1, 8), -1e30, jnp.float32).at[0, 0:NET].set(b_out)
    gold3 = golden_edge.astype(jnp.int32).reshape(B, 1, NPAIR)

    out = _dense_forward(
        amat, s8, x2d, props_pad, wprop_pad, b_prop.reshape(1, 32),
        W_add, b_add.reshape(1, HID), W_in, b_in.reshape(1, HID),
        W_msg, wedge8, W_upd, b_upd.reshape(1, HID),
        W_m1, b_m1.reshape(1, HID), W_m2, b_m2.reshape(1, HID),
        W_m3, b_m3.reshape(1, 2 * HID), wout8, bout8, gold3)
    return out[0, 0] / jnp.float32(B * NODE * NODE)


# speedup vs baseline: 14.0861x; 2.0190x over previous
import jax
import jax.numpy as jnp
from jax import lax
from jax.experimental import pallas as pl
from jax.experimental.pallas import tpu as pltpu
from jax.experimental.pallas import tpu_sc as plsc

B = 32
NODE = 64
NODE_DIM = 160
HID = 64
NET = 5
N = B * NODE
E = 32768
NPAIR = NODE * NODE


_NW = 32
_ROWS = 32
_SLAB = _ROWS * N
_CHUNK = 4096
_NCHUNK = E // _CHUNK


def _sc_body(src_hbm, dst_hbm, ea0, ea1, ea2, ea3, ea4, zro_hbm,
             a_hbm, s_hbm, abuf, sbuf, srcb, dstb, eb0, eb1, eb2, eb3, eb4):
    wid = lax.axis_index("s") * 2 + lax.axis_index("c")

    ones16 = jnp.ones((16,), jnp.float32)
    ea_hbm = (ea0, ea1, ea2, ea3, ea4)
    eabs = (eb0, eb1, eb2, eb3, eb4)

    for p in range(2):
        c = wid + p * _NW
        pltpu.sync_copy(zro_hbm, abuf)
        if p == 0:
            pltpu.sync_copy(zro_hbm.at[pl.ds(0, 8 * NODE)], sbuf)

        for chunk in range(_NCHUNK):
            base = chunk * _CHUNK
            pltpu.sync_copy(src_hbm.at[pl.ds(base, _CHUNK)], srcb)
            pltpu.sync_copy(dst_hbm.at[pl.ds(base, _CHUNK)], dstb)
            if p == 0:
                for k in range(NET):
                    pltpu.sync_copy(ea_hbm[k].at[pl.ds(base, _CHUNK)],
                                    eabs[k])

            def scan_vec(v, _):
                s = srcb[pl.ds(v * 16, 16)]
                d = dstb[pl.ds(v * 16, 16)]
                ok = (s >> 5) == c
                row = jnp.where(ok, s - c * _ROWS, 0)
                plsc.addupdate_scatter(
                    abuf, [row * N + d], ones16, mask=ok)
                if p == 0:
                    oks = (d >> 6) == wid
                    dls = jnp.where(oks, d - wid * NODE, 0)
                    for k in range(NET):
                        vals = eabs[k][pl.ds(v * 16, 16)]
                        plsc.addupdate_scatter(
                            sbuf, [k * NODE + dls], vals, mask=oks)
                return _

            lax.fori_loop(0, _CHUNK // 16, scan_vec, None)

        pltpu.sync_copy(abuf, a_hbm.at[pl.ds(c * _ROWS * N, _SLAB)])

    for k in range(8):
        pltpu.sync_copy(sbuf.at[pl.ds(k * NODE, NODE)],
                        s_hbm.at[pl.ds(k * N + wid * NODE, NODE)])


def _build_adjacency(src, dst, ea_cols):
    mesh = plsc.VectorSubcoreMesh(core_axis_name="c", subcore_axis_name="s")
    f = pl.kernel(
        _sc_body,
        out_type=(
            jax.ShapeDtypeStruct((N * N,), jnp.float32),
            jax.ShapeDtypeStruct((8 * N,), jnp.float32),
        ),
        mesh=mesh,
        compiler_params=pltpu.CompilerParams(
            needs_layout_passes=False, use_tc_tiling_on_sc=False),
        scratch_types=[
            pltpu.VMEM((_SLAB,), jnp.float32),
            pltpu.VMEM((8 * NODE,), jnp.float32),
            pltpu.VMEM((_CHUNK,), jnp.int32),
            pltpu.VMEM((_CHUNK,), jnp.int32),
            pltpu.VMEM((_CHUNK,), jnp.float32),
            pltpu.VMEM((_CHUNK,), jnp.float32),
            pltpu.VMEM((_CHUNK,), jnp.float32),
            pltpu.VMEM((_CHUNK,), jnp.float32),
            pltpu.VMEM((_CHUNK,), jnp.float32),
        ],
    )
    zro = jnp.zeros((_SLAB,), jnp.float32)
    at_flat, st_flat = f(src, dst, ea_cols[0], ea_cols[1], ea_cols[2],
                         ea_cols[3], ea_cols[4], zro)
    return at_flat.reshape(N, N), st_flat.reshape(8, N)


def _dot(a, b):
    return jnp.dot(a, b, precision=lax.Precision.HIGHEST,
                   preferred_element_type=jnp.float32)


def _split(a):
    hi = a.astype(jnp.bfloat16)
    lo = (a - hi.astype(jnp.float32)).astype(jnp.bfloat16)
    return hi, lo


def _dot_bf(a, b):
    return jnp.dot(a, b, preferred_element_type=jnp.float32)


def _dot_sbf(a, b_bf):
    hi, lo = _split(a)
    return _dot_bf(hi, b_bf) + _dot_bf(lo, b_bf)


def _dot3(a, b):
    ahi, alo = _split(a)
    bhi, blo = _split(b)
    return _dot_bf(ahi, bhi) + _dot_bf(alo, bhi) + _dot_bf(ahi, blo)


def _tc_body(at_ref, st_ref, xt_ref, props_ref, wpropc_ref, bpropc_ref,
             waddt_ref, baddc_ref, wint_ref, binc_ref, wmsgt_ref,
             wedge8t_ref, wupdt_ref, bupdc_ref, wm1t_ref, bm1c_ref,
             wm2t_ref, bm2c_ref, wm3t_ref, bm3c_ref, wout8t_ref,
             bout8c_ref, gold_ref, out_ref,
             pt_scr, qt_scr, ht_scr, hhi_scr, hlo_scr, erep_scr, etile_scr):
    pid = pl.program_id(0)

    @pl.when(pid == 0)
    def _encoder():
        ht_scr[...] = jnp.maximum(
            _dot(wint_ref[...], xt_ref[...]) + binc_ref[...], 0.0)
        ct = _dot(wedge8t_ref[...], st_ref[...])
        u1t = wupdt_ref[:, 0:HID]
        u2t = wupdt_ref[:, HID:2 * HID]
        wmsgt = wmsgt_ref[...]
        bupdc = bupdc_ref[...]
        for _ in range(4):
            ht = ht_scr[...]
            hhi = ht.astype(jnp.bfloat16)
            hhi_scr[...] = hhi
            hlo_scr[...] = (ht - hhi.astype(jnp.float32)).astype(jnp.bfloat16)

            def kstep(kb, acc):
                at_blk = at_ref[pl.ds(kb * 128, 128), :]
                return (acc
                        + jnp.dot(hhi_scr[:, pl.ds(kb * 128, 128)], at_blk,
                                  preferred_element_type=jnp.float32)
                        + jnp.dot(hlo_scr[:, pl.ds(kb * 128, 128)], at_blk,
                                  preferred_element_type=jnp.float32))

            gt = lax.fori_loop(0, 16, kstep,
                               jnp.zeros((HID, N), jnp.float32))
            aggt = _dot(wmsgt, gt) + ct
            ht_scr[...] = jnp.maximum(
                _dot(u1t, ht) + _dot(u2t, aggt) + bupdc, 0.0)

        propt = wpropc_ref[...] * props_ref[...] + bpropc_ref[...]
        pvect = _dot(waddt_ref[:, HID:], propt) + baddc_ref[...]
        erep2 = (lax.broadcasted_iota(jnp.int32, (B, N), 0)
                 == lax.broadcasted_iota(jnp.int32, (B, N), 1)
                 // NODE).astype(jnp.float32)
        proppt = _dot(pvect, erep2)
        negt = _dot(waddt_ref[:, 0:HID], ht_scr[...]) + proppt
        pt_scr[...] = _dot(wm1t_ref[:, 0:HID], negt) + bm1c_ref[...]
        qt_scr[...] = _dot(wm1t_ref[:, HID:], negt)
        erep_scr[...] = (lax.broadcasted_iota(jnp.int32, (NODE, NPAIR), 0)
                         == lax.broadcasted_iota(jnp.int32, (NODE, NPAIR), 1)
                         // NODE).astype(jnp.bfloat16)
        etile_scr[...] = (lax.broadcasted_iota(jnp.int32, (NODE, NPAIR), 0)
                          == lax.broadcasted_iota(jnp.int32, (NODE, NPAIR), 1)
                          % NODE).astype(jnp.bfloat16)
        out_ref[...] = jnp.zeros((1, 1), jnp.float32)

    pblk = pt_scr[:, pl.ds((pid // 2) * 128, 128)]
    qblk = qt_scr[:, pl.ds((pid // 2) * 128, 128)]
    odd = (pid % 2) == 1
    pbt = jnp.where(odd, pblk[:, NODE:], pblk[:, 0:NODE])
    qbt = jnp.where(odd, qblk[:, NODE:], qblk[:, 0:NODE])
    h1t = jnp.maximum(
        _dot_sbf(pbt, erep_scr[...]) + _dot_sbf(qbt, etile_scr[...]), 0.0)
    h2t = jnp.maximum(_dot3(wm2t_ref[...], h1t) + bm2c_ref[...], 0.0)
    w35t = _dot(wout8t_ref[...], wm3t_ref[...])
    b35c = _dot(wout8t_ref[...], bm3c_ref[...]) + bout8c_ref[...]
    logitst = _dot3(w35t, h2t) + b35c
    mx = jnp.max(logitst, axis=0, keepdims=True)
    lse = jnp.log(jnp.sum(jnp.exp(logitst - mx), axis=0, keepdims=True)) + mx
    gold = gold_ref[...].reshape(1, NPAIR)
    onehott = (lax.broadcasted_iota(jnp.int32, (8, NPAIR), 0)
               == gold).astype(jnp.float32)
    pick = jnp.sum(logitst * onehott)
    out_ref[...] += jnp.reshape(jnp.sum(lse) - pick, (1, 1))


def _dense_forward(at_bf, st, xt, props_row, wprop_col, bprop_col, waddt,
                   badd_col, wint, bin_col, wmsgt, wedge8t, wupdt, bupd_col,
                   wm1t, bm1_col, wm2t, bm2_col, wm3t, bm3_col, wout8t,
                   bout8_col, gold3, interpret=False):
    full = lambda shape: pl.BlockSpec(shape, lambda b: (0,) * len(shape))
    out = pl.pallas_call(
        _tc_body,
        grid=(B,),
        in_specs=[
            full((N, N)), full((8, N)), full((NODE_DIM, N)),
            full((1, B)), full((B, 1)), full((B, 1)),
            full((HID, 96)), full((HID, 1)),
            full((HID, NODE_DIM)), full((HID, 1)),
            full((HID, HID)), full((HID, 8)),
            full((HID, 2 * HID)), full((HID, 1)),
            full((HID, 2 * HID)), full((HID, 1)),
            full((HID, HID)), full((HID, 1)),
            full((2 * HID, HID)), full((2 * HID, 1)),
            full((8, 2 * HID)), full((8, 1)),
            pl.BlockSpec((1, 1, NPAIR), lambda b: (b, 0, 0)),
        ],
        out_specs=pl.BlockSpec((1, 1), lambda b: (0, 0)),
        out_shape=jax.ShapeDtypeStruct((1, 1), jnp.float32),
        scratch_shapes=[
            pltpu.VMEM((HID, N), jnp.float32),
            pltpu.VMEM((HID, N), jnp.float32),
            pltpu.VMEM((HID, N), jnp.float32),
            pltpu.VMEM((HID, N), jnp.bfloat16),
            pltpu.VMEM((HID, N), jnp.bfloat16),
            pltpu.VMEM((NODE, NPAIR), jnp.bfloat16),
            pltpu.VMEM((NODE, NPAIR), jnp.bfloat16),
        ],
        interpret=interpret,
    )(at_bf, st, xt, props_row, wprop_col, bprop_col, waddt, badd_col,
      wint, bin_col, wmsgt, wedge8t, wupdt, bupd_col, wm1t, bm1_col,
      wm2t, bm2_col, wm3t, bm3_col, wout8t, bout8_col, gold3)
    return out


@jax.jit
def kernel(x, edge_index, edge_attr, pieces, edge_select, golden_edge,
           props, W_in, b_in, W_msg, W_edge, W_upd, b_upd, W_prop, b_prop,
           W_add, b_add, W_m1, b_m1, W_m2, b_m2, W_m3, b_m3, W_out, b_out):
    src = edge_index[0].astype(jnp.int32)
    dst = edge_index[1].astype(jnp.int32)
    ea_cols = [edge_attr[:, k] for k in range(NET)]

    at_mat, st = _build_adjacency(src, dst, ea_cols)
    at_bf = at_mat.astype(jnp.bfloat16)

    xt = x.reshape(N, NODE_DIM).T
    wedge8t = jnp.zeros((HID, 8), jnp.float32).at[:, 0:NET].set(W_edge.T)
    wout8t = jnp.zeros((8, 2 * HID), jnp.float32).at[0:NET, :].set(W_out.T)
    bout8c = jnp.full((8, 1), -1e30, jnp.float32).at[0:NET, 0].set(b_out)
    gold3 = golden_edge.astype(jnp.int32).reshape(B, 1, NPAIR)

    out = _dense_forward(
        at_bf, st, xt, props.reshape(1, B), W_prop.reshape(B, 1),
        b_prop.reshape(B, 1), W_add.T, b_add.reshape(HID, 1),
        W_in.T, b_in.reshape(HID, 1), W_msg.T, wedge8t, W_upd.T,
        b_upd.reshape(HID, 1), W_m1.T, b_m1.reshape(HID, 1), W_m2.T,
        b_m2.reshape(HID, 1), W_m3.T, b_m3.reshape(2 * HID, 1), wout8t,
        bout8c, gold3)
    return out[0, 0] / jnp.float32(B * NODE * NODE)


# speedup vs baseline: 18.1312x; 1.2872x over previous
import jax
import jax.numpy as jnp
from jax import lax
from jax.experimental import pallas as pl
from jax.experimental.pallas import tpu as pltpu
from jax.experimental.pallas import tpu_sc as plsc

B = 32
NODE = 64
NODE_DIM = 160
HID = 64
NET = 5
N = B * NODE
E = 32768
NPAIR = NODE * NODE


_NW = 32
_ROWS = 32
_SLAB = _ROWS * N
_CHUNK = 4096
_NCHUNK = E // _CHUNK


def _sc_body(src_hbm, dst_hbm, ea0, ea1, ea2, ea3, ea4,
             a_hbm, s_hbm, abuf, sbuf, srcb, dstb,
             eb0, eb1, eb2, eb3, eb4, sem0, sem1):
    wid = lax.axis_index("s") * 2 + lax.axis_index("c")

    zeros16 = jnp.zeros((16,), jnp.float32)
    ones16 = jnp.ones((16,), jnp.float32)
    ea_hbm = (ea0, ea1, ea2, ea3, ea4)
    eabs = (eb0, eb1, eb2, eb3, eb4)
    sems = (sem0, sem1)

    def zero_a(i, _):
        for u in range(4):
            abuf[pl.ds(i * 64 + u * 16, 16)] = zeros16
        return _

    def issue(chunk, p):
        par = chunk % 2
        base = chunk * _CHUNK
        half = pl.ds(base, _CHUNK)
        dsts = pl.ds(par * _CHUNK, _CHUNK)
        cps = [pltpu.make_async_copy(src_hbm.at[half], srcb.at[dsts],
                                     sems[par]),
               pltpu.make_async_copy(dst_hbm.at[half], dstb.at[dsts],
                                     sems[par])]
        if p == 0:
            for k in range(NET):
                cps.append(pltpu.make_async_copy(
                    ea_hbm[k].at[half], eabs[k].at[dsts], sems[par]))
        for cp in cps:
            cp.start()
        return cps

    for p in range(2):
        c = wid + p * _NW
        pend = issue(0, p)
        lax.fori_loop(0, _SLAB // 64, zero_a, None)
        if p == 0:
            def zero_s(i, _):
                sbuf[pl.ds(i * 16, 16)] = zeros16
                return _
            lax.fori_loop(0, (8 * NODE) // 16, zero_s, None)

        for chunk in range(_NCHUNK):
            par = chunk % 2
            boff = par * _CHUNK
            for cp in pend:
                cp.wait()
            if chunk + 1 < _NCHUNK:
                pend = issue(chunk + 1, p)

            def scan_vec(v, _):
                for u in range(4):
                    o = pl.ds(boff + v * 64 + u * 16, 16)
                    s = srcb[o]
                    d = dstb[o]
                    ok = (s >> 5) == c
                    row = jnp.where(ok, s - c * _ROWS, 0)
                    plsc.addupdate_scatter(
                        abuf, [row * N + d], ones16, mask=ok)
                    if p == 0:
                        oks = (d >> 6) == wid
                        dls = jnp.where(oks, d - wid * NODE, 0)
                        for k in range(NET):
                            plsc.addupdate_scatter(
                                sbuf, [k * NODE + dls], eabs[k][o], mask=oks)
                return _

            lax.fori_loop(0, _CHUNK // 64, scan_vec, None)

        pltpu.sync_copy(abuf, a_hbm.at[pl.ds(c * _ROWS * N, _SLAB)])

    for k in range(8):
        pltpu.sync_copy(sbuf.at[pl.ds(k * NODE, NODE)],
                        s_hbm.at[pl.ds(k * N + wid * NODE, NODE)])


def _build_adjacency(src, dst, ea_cols):
    mesh = plsc.VectorSubcoreMesh(core_axis_name="c", subcore_axis_name="s")
    f = pl.kernel(
        _sc_body,
        out_type=(
            jax.ShapeDtypeStruct((N * N,), jnp.float32),
            jax.ShapeDtypeStruct((8 * N,), jnp.float32),
        ),
        mesh=mesh,
        compiler_params=pltpu.CompilerParams(
            needs_layout_passes=False, use_tc_tiling_on_sc=False),
        scratch_types=[
            pltpu.VMEM((_SLAB,), jnp.float32),
            pltpu.VMEM((8 * NODE,), jnp.float32),
            pltpu.VMEM((2 * _CHUNK,), jnp.int32),
            pltpu.VMEM((2 * _CHUNK,), jnp.int32),
            pltpu.VMEM((2 * _CHUNK,), jnp.float32),
            pltpu.VMEM((2 * _CHUNK,), jnp.float32),
            pltpu.VMEM((2 * _CHUNK,), jnp.float32),
            pltpu.VMEM((2 * _CHUNK,), jnp.float32),
            pltpu.VMEM((2 * _CHUNK,), jnp.float32),
            pltpu.SemaphoreType.DMA,
            pltpu.SemaphoreType.DMA,
        ],
    )
    at_flat, st_flat = f(src, dst, ea_cols[0], ea_cols[1], ea_cols[2],
                         ea_cols[3], ea_cols[4])
    return at_flat.reshape(N, N), st_flat.reshape(8, N)


def _dot(a, b):
    return jnp.dot(a, b, precision=lax.Precision.HIGHEST,
                   preferred_element_type=jnp.float32)


def _split(a):
    hi = a.astype(jnp.bfloat16)
    lo = (a - hi.astype(jnp.float32)).astype(jnp.bfloat16)
    return hi, lo


def _dot_bf(a, b):
    return jnp.dot(a, b, preferred_element_type=jnp.float32)


def _dot_sbf(a, b_bf):
    hi, lo = _split(a)
    return _dot_bf(hi, b_bf) + _dot_bf(lo, b_bf)


def _dot3(a, b):
    ahi, alo = _split(a)
    bhi, blo = _split(b)
    return _dot_bf(ahi, bhi) + _dot_bf(alo, bhi) + _dot_bf(ahi, blo)


def _tc_body(at_ref, st_ref, xt_ref, props_ref, wpropc_ref, bpropc_ref,
             waddt_ref, baddc_ref, wint_ref, binc_ref, wmsgt_ref,
             wedge8t_ref, wupdt_ref, bupdc_ref, wm1t_ref, bm1c_ref,
             wm2t_ref, bm2c_ref, wm3t_ref, bm3c_ref, wout8t_ref,
             bout8c_ref, gold_ref, out_ref,
             pt_scr, qt_scr, ht_scr, hhi_scr, hlo_scr, erep_scr, etile_scr):
    pid = pl.program_id(0)

    @pl.when(pid == 0)
    def _encoder():
        ht_scr[...] = jnp.maximum(
            _dot(wint_ref[...], xt_ref[...]) + binc_ref[...], 0.0)
        ct = _dot(wedge8t_ref[...], st_ref[...])
        u1t = wupdt_ref[:, 0:HID]
        u2t = wupdt_ref[:, HID:2 * HID]
        wmsgt = wmsgt_ref[...]
        bupdc = bupdc_ref[...]
        for _ in range(4):
            ht = ht_scr[...]
            hhi = ht.astype(jnp.bfloat16)
            hhi_scr[...] = hhi
            hlo_scr[...] = (ht - hhi.astype(jnp.float32)).astype(jnp.bfloat16)

            def kstep(kb, acc):
                at_blk = at_ref[pl.ds(kb * 128, 128), :]
                return (acc
                        + jnp.dot(hhi_scr[:, pl.ds(kb * 128, 128)], at_blk,
                                  preferred_element_type=jnp.float32)
                        + jnp.dot(hlo_scr[:, pl.ds(kb * 128, 128)], at_blk,
                                  preferred_element_type=jnp.float32))

            gt = lax.fori_loop(0, 16, kstep,
                               jnp.zeros((HID, N), jnp.float32))
            aggt = _dot(wmsgt, gt) + ct
            ht_scr[...] = jnp.maximum(
                _dot(u1t, ht) + _dot(u2t, aggt) + bupdc, 0.0)

        propt = wpropc_ref[...] * props_ref[...] + bpropc_ref[...]
        pvect = _dot(waddt_ref[:, HID:], propt) + baddc_ref[...]
        erep2 = (lax.broadcasted_iota(jnp.int32, (B, N), 0)
                 == lax.broadcasted_iota(jnp.int32, (B, N), 1)
                 // NODE).astype(jnp.float32)
        proppt = _dot(pvect, erep2)
        negt = _dot(waddt_ref[:, 0:HID], ht_scr[...]) + proppt
        pt_scr[...] = _dot(wm1t_ref[:, 0:HID], negt) + bm1c_ref[...]
        qt_scr[...] = _dot(wm1t_ref[:, HID:], negt)
        erep_scr[...] = (lax.broadcasted_iota(jnp.int32, (NODE, NPAIR), 0)
                         == lax.broadcasted_iota(jnp.int32, (NODE, NPAIR), 1)
                         // NODE).astype(jnp.bfloat16)
        etile_scr[...] = (lax.broadcasted_iota(jnp.int32, (NODE, NPAIR), 0)
                          == lax.broadcasted_iota(jnp.int32, (NODE, NPAIR), 1)
                          % NODE).astype(jnp.bfloat16)
        out_ref[...] = jnp.zeros((1, 1), jnp.float32)

    pblk = pt_scr[:, pl.ds((pid // 2) * 128, 128)]
    qblk = qt_scr[:, pl.ds((pid // 2) * 128, 128)]
    odd = (pid % 2) == 1
    pbt = jnp.where(odd, pblk[:, NODE:], pblk[:, 0:NODE])
    qbt = jnp.where(odd, qblk[:, NODE:], qblk[:, 0:NODE])
    h1t = jnp.maximum(
        _dot_sbf(pbt, erep_scr[...]) + _dot_sbf(qbt, etile_scr[...]), 0.0)
    h2t = jnp.maximum(_dot3(wm2t_ref[...], h1t) + bm2c_ref[...], 0.0)
    w35t = _dot(wout8t_ref[...], wm3t_ref[...])
    b35c = _dot(wout8t_ref[...], bm3c_ref[...]) + bout8c_ref[...]
    logitst = _dot3(w35t, h2t) + b35c
    mx = jnp.max(logitst, axis=0, keepdims=True)
    lse = jnp.log(jnp.sum(jnp.exp(logitst - mx), axis=0, keepdims=True)) + mx
    gold = gold_ref[...].reshape(1, NPAIR)
    onehott = (lax.broadcasted_iota(jnp.int32, (8, NPAIR), 0)
               == gold).astype(jnp.float32)
    pick = jnp.sum(logitst * onehott)
    out_ref[...] += jnp.reshape(jnp.sum(lse) - pick, (1, 1))


def _dense_forward(at_bf, st, xt, props_row, wprop_col, bprop_col, waddt,
                   badd_col, wint, bin_col, wmsgt, wedge8t, wupdt, bupd_col,
                   wm1t, bm1_col, wm2t, bm2_col, wm3t, bm3_col, wout8t,
                   bout8_col, gold3, interpret=False):
    full = lambda shape: pl.BlockSpec(shape, lambda b: (0,) * len(shape))
    out = pl.pallas_call(
        _tc_body,
        grid=(B,),
        in_specs=[
            full((N, N)), full((8, N)), full((NODE_DIM, N)),
            full((1, B)), full((B, 1)), full((B, 1)),
            full((HID, 96)), full((HID, 1)),
            full((HID, NODE_DIM)), full((HID, 1)),
            full((HID, HID)), full((HID, 8)),
            full((HID, 2 * HID)), full((HID, 1)),
            full((HID, 2 * HID)), full((HID, 1)),
            full((HID, HID)), full((HID, 1)),
            full((2 * HID, HID)), full((2 * HID, 1)),
            full((8, 2 * HID)), full((8, 1)),
            pl.BlockSpec((1, 1, NPAIR), lambda b: (b, 0, 0)),
        ],
        out_specs=pl.BlockSpec((1, 1), lambda b: (0, 0)),
        out_shape=jax.ShapeDtypeStruct((1, 1), jnp.float32),
        scratch_shapes=[
            pltpu.VMEM((HID, N), jnp.float32),
            pltpu.VMEM((HID, N), jnp.float32),
            pltpu.VMEM((HID, N), jnp.float32),
            pltpu.VMEM((HID, N), jnp.bfloat16),
            pltpu.VMEM((HID, N), jnp.bfloat16),
            pltpu.VMEM((NODE, NPAIR), jnp.bfloat16),
            pltpu.VMEM((NODE, NPAIR), jnp.bfloat16),
        ],
        interpret=interpret,
    )(at_bf, st, xt, props_row, wprop_col, bprop_col, waddt, badd_col,
      wint, bin_col, wmsgt, wedge8t, wupdt, bupd_col, wm1t, bm1_col,
      wm2t, bm2_col, wm3t, bm3_col, wout8t, bout8_col, gold3)
    return out


@jax.jit
def kernel(x, edge_index, edge_attr, pieces, edge_select, golden_edge,
           props, W_in, b_in, W_msg, W_edge, W_upd, b_upd, W_prop, b_prop,
           W_add, b_add, W_m1, b_m1, W_m2, b_m2, W_m3, b_m3, W_out, b_out):
    src = edge_index[0].astype(jnp.int32)
    dst = edge_index[1].astype(jnp.int32)
    ea_cols = [edge_attr[:, k] for k in range(NET)]

    at_mat, st = _build_adjacency(src, dst, ea_cols)
    at_bf = at_mat.astype(jnp.bfloat16)

    xt = x.reshape(N, NODE_DIM).T
    wedge8t = jnp.zeros((HID, 8), jnp.float32).at[:, 0:NET].set(W_edge.T)
    wout8t = jnp.zeros((8, 2 * HID), jnp.float32).at[0:NET, :].set(W_out.T)
    bout8c = jnp.full((8, 1), -1e30, jnp.float32).at[0:NET, 0].set(b_out)
    gold3 = golden_edge.astype(jnp.int32).reshape(B, 1, NPAIR)

    out = _dense_forward(
        at_bf, st, xt, props.reshape(1, B), W_prop.reshape(B, 1),
        b_prop.reshape(B, 1), W_add.T, b_add.reshape(HID, 1),
        W_in.T, b_in.reshape(HID, 1), W_msg.T, wedge8t, W_upd.T,
        b_upd.reshape(HID, 1), W_m1.T, b_m1.reshape(HID, 1), W_m2.T,
        b_m2.reshape(HID, 1), W_m3.T, b_m3.reshape(2 * HID, 1), wout8t,
        bout8c, gold3)
    return out[0, 0] / jnp.float32(B * NODE * NODE)


# speedup vs baseline: 19.9039x; 1.0978x over previous
import jax
import jax.numpy as jnp
from jax import lax
from jax.experimental import pallas as pl
from jax.experimental.pallas import tpu as pltpu
from jax.experimental.pallas import tpu_sc as plsc

B = 32
NODE = 64
NODE_DIM = 160
HID = 64
NET = 5
N = B * NODE
E = 32768
NPAIR = NODE * NODE


_NW = 32
_ROWS = 32
_SLAB = _ROWS * N
_CHUNK = 4096
_NCHUNK = E // _CHUNK


def _sc_body(src_hbm, dst_hbm, ea0, ea1, ea2, ea3, ea4,
             a_hbm, s_hbm, abuf, sbuf, srcb, dstb,
             eb0, eb1, eb2, eb3, eb4, sem0, sem1):
    wid = lax.axis_index("s") * 2 + lax.axis_index("c")

    zeros16 = jnp.zeros((16,), jnp.float32)
    ones16 = jnp.ones((16,), jnp.float32)
    ea_hbm = (ea0, ea1, ea2, ea3, ea4)
    eabs = (eb0, eb1, eb2, eb3, eb4)
    sems = (sem0, sem1)

    def zero_a(i, _):
        for u in range(4):
            abuf[pl.ds(i * 64 + u * 16, 16)] = zeros16
        return _

    def issue(chunk, p):
        par = chunk % 2
        base = chunk * _CHUNK
        half = pl.ds(base, _CHUNK)
        dsts = pl.ds(par * _CHUNK, _CHUNK)
        cps = [pltpu.make_async_copy(src_hbm.at[half], srcb.at[dsts],
                                     sems[par]),
               pltpu.make_async_copy(dst_hbm.at[half], dstb.at[dsts],
                                     sems[par])]
        if p == 0:
            for k in range(NET):
                cps.append(pltpu.make_async_copy(
                    ea_hbm[k].at[half], eabs[k].at[dsts], sems[par]))
        for cp in cps:
            cp.start()
        return cps

    for p in range(2):
        c = wid + p * _NW
        pend = issue(0, p)
        lax.fori_loop(0, _SLAB // 64, zero_a, None)
        if p == 0:
            def zero_s(i, _):
                sbuf[pl.ds(i * 16, 16)] = zeros16
                return _
            lax.fori_loop(0, (8 * NODE) // 16, zero_s, None)

        for chunk in range(_NCHUNK):
            par = chunk % 2
            boff = par * _CHUNK
            for cp in pend:
                cp.wait()
            if chunk + 1 < _NCHUNK:
                pend = issue(chunk + 1, p)

            def scan_vec(v, _):
                for u in range(4):
                    o = pl.ds(boff + v * 64 + u * 16, 16)
                    s = srcb[o]
                    d = dstb[o]
                    ok = (s >> 5) == c
                    row = jnp.where(ok, s - c * _ROWS, 0)
                    plsc.addupdate_scatter(
                        abuf, [row * N + d], ones16, mask=ok)
                    if p == 0:
                        oks = (d >> 6) == wid
                        dls = jnp.where(oks, d - wid * NODE, 0)
                        for k in range(NET):
                            plsc.addupdate_scatter(
                                sbuf, [k * NODE + dls], eabs[k][o], mask=oks)
                return _

            lax.fori_loop(0, _CHUNK // 64, scan_vec, None)

        pltpu.sync_copy(abuf, a_hbm.at[pl.ds(c * _ROWS * N, _SLAB)])

    for k in range(8):
        pltpu.sync_copy(sbuf.at[pl.ds(k * NODE, NODE)],
                        s_hbm.at[pl.ds(k * N + wid * NODE, NODE)])


def _build_adjacency(src, dst, ea_cols):
    mesh = plsc.VectorSubcoreMesh(core_axis_name="c", subcore_axis_name="s")
    f = pl.kernel(
        _sc_body,
        out_type=(
            jax.ShapeDtypeStruct((N * N,), jnp.float32),
            jax.ShapeDtypeStruct((8 * N,), jnp.float32),
        ),
        mesh=mesh,
        compiler_params=pltpu.CompilerParams(
            needs_layout_passes=False, use_tc_tiling_on_sc=False),
        scratch_types=[
            pltpu.VMEM((_SLAB,), jnp.float32),
            pltpu.VMEM((8 * NODE,), jnp.float32),
            pltpu.VMEM((2 * _CHUNK,), jnp.int32),
            pltpu.VMEM((2 * _CHUNK,), jnp.int32),
            pltpu.VMEM((2 * _CHUNK,), jnp.float32),
            pltpu.VMEM((2 * _CHUNK,), jnp.float32),
            pltpu.VMEM((2 * _CHUNK,), jnp.float32),
            pltpu.VMEM((2 * _CHUNK,), jnp.float32),
            pltpu.VMEM((2 * _CHUNK,), jnp.float32),
            pltpu.SemaphoreType.DMA,
            pltpu.SemaphoreType.DMA,
        ],
    )
    at_flat, st_flat = f(src, dst, ea_cols[0], ea_cols[1], ea_cols[2],
                         ea_cols[3], ea_cols[4])
    return at_flat.reshape(N, N), st_flat.reshape(8, N)


def _dot(a, b):
    return jnp.dot(a, b, precision=lax.Precision.HIGHEST,
                   preferred_element_type=jnp.float32)


def _split(a):
    hi = a.astype(jnp.bfloat16)
    lo = (a - hi.astype(jnp.float32)).astype(jnp.bfloat16)
    return hi, lo


def _dot_bf(a, b):
    return jnp.dot(a, b, preferred_element_type=jnp.float32)


def _dot_sbf(a, b_bf):
    hi, lo = _split(a)
    return _dot_bf(hi, b_bf) + _dot_bf(lo, b_bf)


def _dot3(a, b):
    ahi, alo = _split(a)
    bhi, blo = _split(b)
    return _dot_bf(ahi, bhi) + _dot_bf(alo, bhi) + _dot_bf(ahi, blo)


def _tc_body(at_ref, st_ref, xt_ref, props_ref, wpropc_ref, bpropc_ref,
             waddt_ref, baddc_ref, wint_ref, binc_ref, wmsgt_ref,
             wedge8t_ref, wupdt_ref, bupdc_ref, wm1t_ref, bm1c_ref,
             wm2t_ref, bm2c_ref, wm3t_ref, bm3c_ref, wout8t_ref,
             bout8c_ref, gold_ref, out_ref,
             ptq_scr, ht_scr, hhi_scr, hlo_scr, ee_scr, w35_scr, b35_scr):
    pid = pl.program_id(0)

    @pl.when(pid == 0)
    def _encoder():
        ht_scr[...] = jnp.maximum(
            _dot3(wint_ref[...], xt_ref[...]) + binc_ref[...], 0.0)
        ct = _dot3(wedge8t_ref[...], st_ref[...])
        u1t = wupdt_ref[:, 0:HID]
        u2t = wupdt_ref[:, HID:2 * HID]
        wmsgt = wmsgt_ref[...]
        bupdc = bupdc_ref[...]
        for _ in range(4):
            ht = ht_scr[...]
            hhi = ht.astype(jnp.bfloat16)
            hhi_scr[...] = hhi
            hlo_scr[...] = (ht - hhi.astype(jnp.float32)).astype(jnp.bfloat16)

            def kstep(kb, acc):
                at_blk = at_ref[pl.ds(kb * 128, 128), :]
                return (acc
                        + jnp.dot(hhi_scr[:, pl.ds(kb * 128, 128)], at_blk,
                                  preferred_element_type=jnp.float32)
                        + jnp.dot(hlo_scr[:, pl.ds(kb * 128, 128)], at_blk,
                                  preferred_element_type=jnp.float32))

            gt = lax.fori_loop(0, 16, kstep,
                               jnp.zeros((HID, N), jnp.float32))
            aggt = _dot3(wmsgt, gt) + ct
            ht_scr[...] = jnp.maximum(
                _dot3(u1t, ht) + _dot3(u2t, aggt) + bupdc, 0.0)

        propt = wpropc_ref[...] * props_ref[...] + bpropc_ref[...]
        pvect = _dot3(waddt_ref[:, HID:], propt) + baddc_ref[...]
        erep2 = (lax.broadcasted_iota(jnp.int32, (B, N), 0)
                 == lax.broadcasted_iota(jnp.int32, (B, N), 1)
                 // NODE).astype(jnp.bfloat16)
        proppt = _dot_sbf(pvect, erep2)
        negt = _dot3(waddt_ref[:, 0:HID], ht_scr[...]) + proppt
        pt = _dot3(wm1t_ref[:, 0:HID], negt) + bm1c_ref[...]
        qt = _dot3(wm1t_ref[:, HID:], negt)
        for b in range(B):
            ptq_scr[:, b * 128:b * 128 + NODE] = pt[:, b * NODE:(b + 1) * NODE]
            ptq_scr[:, b * 128 + NODE:(b + 1) * 128] = (
                qt[:, b * NODE:(b + 1) * NODE])
        rr = lax.broadcasted_iota(jnp.int32, (2 * NODE, NPAIR), 0)
        pp = lax.broadcasted_iota(jnp.int32, (2 * NODE, NPAIR), 1)
        e1 = ((pp // NODE) == rr).astype(jnp.float32)
        e2 = ((pp % NODE) == (rr - NODE)).astype(jnp.float32)
        ee_scr[...] = jnp.where(rr < NODE, e1, e2).astype(jnp.bfloat16)
        w35_scr[...] = _dot3(wout8t_ref[...], wm3t_ref[...])
        b35_scr[...] = _dot3(wout8t_ref[...], bm3c_ref[...]) + bout8c_ref[...]
        out_ref[...] = jnp.zeros((1, 1), jnp.float32)

    pq = ptq_scr[:, pl.ds(pid * 128, 128)]
    h1t = jnp.maximum(_dot_sbf(pq, ee_scr[...]), 0.0)
    h2t = jnp.maximum(_dot3(wm2t_ref[...], h1t) + bm2c_ref[...], 0.0)
    logitst = _dot3(w35_scr[...], h2t) + b35_scr[...]
    mx = jnp.max(logitst, axis=0, keepdims=True)
    lse = jnp.log(jnp.sum(jnp.exp(logitst - mx), axis=0, keepdims=True)) + mx
    gold = gold_ref[...].reshape(1, NPAIR)
    onehott = (lax.broadcasted_iota(jnp.int32, (8, NPAIR), 0)
               == gold).astype(jnp.float32)
    pick = jnp.sum(logitst * onehott)
    out_ref[...] += jnp.reshape(jnp.sum(lse) - pick, (1, 1))


def _dense_forward(at_bf, st, xt, props_row, wprop_col, bprop_col, waddt,
                   badd_col, wint, bin_col, wmsgt, wedge8t, wupdt, bupd_col,
                   wm1t, bm1_col, wm2t, bm2_col, wm3t, bm3_col, wout8t,
                   bout8_col, gold3, interpret=False):
    full = lambda shape: pl.BlockSpec(shape, lambda b: (0,) * len(shape))
    out = pl.pallas_call(
        _tc_body,
        grid=(B,),
        in_specs=[
            full((N, N)), full((8, N)), full((NODE_DIM, N)),
            full((1, B)), full((B, 1)), full((B, 1)),
            full((HID, 96)), full((HID, 1)),
            full((HID, NODE_DIM)), full((HID, 1)),
            full((HID, HID)), full((HID, 8)),
            full((HID, 2 * HID)), full((HID, 1)),
            full((HID, 2 * HID)), full((HID, 1)),
            full((HID, HID)), full((HID, 1)),
            full((2 * HID, HID)), full((2 * HID, 1)),
            full((8, 2 * HID)), full((8, 1)),
            pl.BlockSpec((1, 1, NPAIR), lambda b: (b, 0, 0)),
        ],
        out_specs=pl.BlockSpec((1, 1), lambda b: (0, 0)),
        out_shape=jax.ShapeDtypeStruct((1, 1), jnp.float32),
        scratch_shapes=[
            pltpu.VMEM((HID, 2 * N), jnp.float32),
            pltpu.VMEM((HID, N), jnp.float32),
            pltpu.VMEM((HID, N), jnp.bfloat16),
            pltpu.VMEM((HID, N), jnp.bfloat16),
            pltpu.VMEM((2 * NODE, NPAIR), jnp.bfloat16),
            pltpu.VMEM((8, HID), jnp.float32),
            pltpu.VMEM((8, 1), jnp.float32),
        ],
        interpret=interpret,
    )(at_bf, st, xt, props_row, wprop_col, bprop_col, waddt, badd_col,
      wint, bin_col, wmsgt, wedge8t, wupdt, bupd_col, wm1t, bm1_col,
      wm2t, bm2_col, wm3t, bm3_col, wout8t, bout8_col, gold3)
    return out


@jax.jit
def kernel(x, edge_index, edge_attr, pieces, edge_select, golden_edge,
           props, W_in, b_in, W_msg, W_edge, W_upd, b_upd, W_prop, b_prop,
           W_add, b_add, W_m1, b_m1, W_m2, b_m2, W_m3, b_m3, W_out, b_out):
    src = edge_index[0].astype(jnp.int32)
    dst = edge_index[1].astype(jnp.int32)
    ea_cols = [edge_attr[:, k] for k in range(NET)]

    at_mat, st = _build_adjacency(src, dst, ea_cols)
    at_bf = at_mat.astype(jnp.bfloat16)

    xt = x.reshape(N, NODE_DIM).T
    wedge8t = jnp.zeros((HID, 8), jnp.float32).at[:, 0:NET].set(W_edge.T)
    wout8t = jnp.zeros((8, 2 * HID), jnp.float32).at[0:NET, :].set(W_out.T)
    bout8c = jnp.full((8, 1), -1e30, jnp.float32).at[0:NET, 0].set(b_out)
    gold3 = golden_edge.astype(jnp.int32).reshape(B, 1, NPAIR)

    out = _dense_forward(
        at_bf, st, xt, props.reshape(1, B), W_prop.reshape(B, 1),
        b_prop.reshape(B, 1), W_add.T, b_add.reshape(HID, 1),
        W_in.T, b_in.reshape(HID, 1), W_msg.T, wedge8t, W_upd.T,
        b_upd.reshape(HID, 1), W_m1.T, b_m1.reshape(HID, 1), W_m2.T,
        b_m2.reshape(HID, 1), W_m3.T, b_m3.reshape(2 * HID, 1), wout8t,
        bout8c, gold3)
    return out[0, 0] / jnp.float32(B * NODE * NODE)


# speedup vs baseline: 20.2227x; 1.0160x over previous
import jax
import jax.numpy as jnp
from jax import lax
from jax.experimental import pallas as pl
from jax.experimental.pallas import tpu as pltpu
from jax.experimental.pallas import tpu_sc as plsc

B = 32
NODE = 64
NODE_DIM = 160
HID = 64
NET = 5
N = B * NODE
E = 32768
NPAIR = NODE * NODE


_NW = 32
_ROWS = 32
_SLAB = _ROWS * N
_CHUNK = 4096
_NCHUNK = E // _CHUNK


def _sc_body(src_hbm, dst_hbm, ea0, ea1, ea2, ea3, ea4,
             a_hbm, s_hbm, abuf, sbuf, srcb, dstb,
             eb0, eb1, eb2, eb3, eb4, sem0, sem1):
    wid = lax.axis_index("s") * 2 + lax.axis_index("c")

    zeros16 = jnp.zeros((16,), jnp.float32)
    ones16 = jnp.ones((16,), jnp.float32)
    ea_hbm = (ea0, ea1, ea2, ea3, ea4)
    eabs = (eb0, eb1, eb2, eb3, eb4)
    sems = (sem0, sem1)

    def zero_a(i, _):
        for u in range(4):
            abuf[pl.ds(i * 64 + u * 16, 16)] = zeros16
        return _

    def issue(chunk, p):
        par = chunk % 2
        base = chunk * _CHUNK
        half = pl.ds(base, _CHUNK)
        dsts = pl.ds(par * _CHUNK, _CHUNK)
        cps = [pltpu.make_async_copy(src_hbm.at[half], srcb.at[dsts],
                                     sems[par]),
               pltpu.make_async_copy(dst_hbm.at[half], dstb.at[dsts],
                                     sems[par])]
        if p == 0:
            for k in range(NET):
                cps.append(pltpu.make_async_copy(
                    ea_hbm[k].at[half], eabs[k].at[dsts], sems[par]))
        for cp in cps:
            cp.start()
        return cps

    for p in range(2):
        c = wid + p * _NW
        pend = issue(0, p)
        lax.fori_loop(0, _SLAB // 64, zero_a, None)
        if p == 0:
            def zero_s(i, _):
                sbuf[pl.ds(i * 16, 16)] = zeros16
                return _
            lax.fori_loop(0, (8 * NODE) // 16, zero_s, None)

        for chunk in range(_NCHUNK):
            par = chunk % 2
            boff = par * _CHUNK
            for cp in pend:
                cp.wait()
            if chunk + 1 < _NCHUNK:
                pend = issue(chunk + 1, p)

            def scan_vec(v, _):
                for u in range(8):
                    o = pl.ds(boff + v * 128 + u * 16, 16)
                    s = srcb[o]
                    d = dstb[o]
                    ok = (s >> 5) == c
                    row = jnp.where(ok, s - c * _ROWS, 0)
                    plsc.addupdate_scatter(
                        abuf, [row * N + d], ones16, mask=ok)
                    if p == 0:
                        oks = (d >> 6) == wid
                        dls = jnp.where(oks, d - wid * NODE, 0)
                        for k in range(NET):
                            plsc.addupdate_scatter(
                                sbuf, [k * NODE + dls], eabs[k][o], mask=oks)
                return _

            lax.fori_loop(0, _CHUNK // 128, scan_vec, None)

        pltpu.sync_copy(abuf, a_hbm.at[pl.ds(c * _ROWS * N, _SLAB)])

    for k in range(8):
        pltpu.sync_copy(sbuf.at[pl.ds(k * NODE, NODE)],
                        s_hbm.at[pl.ds(k * N + wid * NODE, NODE)])


def _build_adjacency(src, dst, ea_cols):
    mesh = plsc.VectorSubcoreMesh(core_axis_name="c", subcore_axis_name="s")
    f = pl.kernel(
        _sc_body,
        out_type=(
            jax.ShapeDtypeStruct((N * N,), jnp.float32),
            jax.ShapeDtypeStruct((8 * N,), jnp.float32),
        ),
        mesh=mesh,
        compiler_params=pltpu.CompilerParams(
            needs_layout_passes=False, use_tc_tiling_on_sc=False),
        scratch_types=[
            pltpu.VMEM((_SLAB,), jnp.float32),
            pltpu.VMEM((8 * NODE,), jnp.float32),
            pltpu.VMEM((2 * _CHUNK,), jnp.int32),
            pltpu.VMEM((2 * _CHUNK,), jnp.int32),
            pltpu.VMEM((2 * _CHUNK,), jnp.float32),
            pltpu.VMEM((2 * _CHUNK,), jnp.float32),
            pltpu.VMEM((2 * _CHUNK,), jnp.float32),
            pltpu.VMEM((2 * _CHUNK,), jnp.float32),
            pltpu.VMEM((2 * _CHUNK,), jnp.float32),
            pltpu.SemaphoreType.DMA,
            pltpu.SemaphoreType.DMA,
        ],
    )
    at_flat, st_flat = f(src, dst, ea_cols[0], ea_cols[1], ea_cols[2],
                         ea_cols[3], ea_cols[4])
    return at_flat, st_flat.reshape(8, N)


def _dot(a, b):
    return jnp.dot(a, b, precision=lax.Precision.HIGHEST,
                   preferred_element_type=jnp.float32)


def _split(a):
    hi = a.astype(jnp.bfloat16)
    lo = (a - hi.astype(jnp.float32)).astype(jnp.bfloat16)
    return hi, lo


def _dot_bf(a, b):
    return jnp.dot(a, b, preferred_element_type=jnp.float32)


def _dot_sbf(a, b_bf):
    hi, lo = _split(a)
    return _dot_bf(hi, b_bf) + _dot_bf(lo, b_bf)


def _dot3(a, b):
    ahi, alo = _split(a)
    bhi, blo = _split(b)
    return _dot_bf(ahi, bhi) + _dot_bf(alo, bhi) + _dot_bf(ahi, blo)


def _tc_body(at_ref, st_ref, xt_ref, props_ref, wpropc_ref, bpropc_ref,
             waddt_ref, baddc_ref, wint_ref, binc_ref, wmsgt_ref,
             wedge8t_ref, wupdt_ref, bupdc_ref, wm1t_ref, bm1c_ref,
             wm2t_ref, bm2c_ref, wm3t_ref, bm3c_ref, wout8t_ref,
             bout8c_ref, gold_ref, out_ref,
             ptq_scr, ht_scr, hhi_scr, hlo_scr, ee_scr, w35_scr, b35_scr,
             at_scr):
    pid = pl.program_id(0)

    @pl.when(pid == 0)
    def _encoder():
        for kb in range(32):
            at_scr[pl.ds(kb * 64, 64), :] = (
                at_ref[pl.ds(kb * 64, 64), :].astype(jnp.bfloat16))
        ht_scr[...] = jnp.maximum(
            _dot3(wint_ref[...], xt_ref[...]) + binc_ref[...], 0.0)
        ct = _dot3(wedge8t_ref[...], st_ref[...])
        u1t = wupdt_ref[:, 0:HID]
        u2t = wupdt_ref[:, HID:2 * HID]
        wmsgt = wmsgt_ref[...]
        bupdc = bupdc_ref[...]
        for _ in range(4):
            ht = ht_scr[...]
            hhi = ht.astype(jnp.bfloat16)
            hhi_scr[...] = hhi
            hlo_scr[...] = (ht - hhi.astype(jnp.float32)).astype(jnp.bfloat16)

            def kstep(kb, acc):
                at_blk = at_scr[pl.ds(kb * 128, 128), :]
                return (acc
                        + jnp.dot(hhi_scr[:, pl.ds(kb * 128, 128)], at_blk,
                                  preferred_element_type=jnp.float32)
                        + jnp.dot(hlo_scr[:, pl.ds(kb * 128, 128)], at_blk,
                                  preferred_element_type=jnp.float32))

            gt = lax.fori_loop(0, 16, kstep,
                               jnp.zeros((HID, N), jnp.float32))
            aggt = _dot3(wmsgt, gt) + ct
            ht_scr[...] = jnp.maximum(
                _dot3(u1t, ht) + _dot3(u2t, aggt) + bupdc, 0.0)

        propt = wpropc_ref[...] * props_ref[...] + bpropc_ref[...]
        pvect = _dot3(waddt_ref[:, HID:], propt) + baddc_ref[...]
        erep2 = (lax.broadcasted_iota(jnp.int32, (B, N), 0)
                 == lax.broadcasted_iota(jnp.int32, (B, N), 1)
                 // NODE).astype(jnp.bfloat16)
        proppt = _dot_sbf(pvect, erep2)
        negt = _dot3(waddt_ref[:, 0:HID], ht_scr[...]) + proppt
        pt = _dot3(wm1t_ref[:, 0:HID], negt) + bm1c_ref[...]
        qt = _dot3(wm1t_ref[:, HID:], negt)
        for b in range(B):
            ptq_scr[:, b * 128:b * 128 + NODE] = pt[:, b * NODE:(b + 1) * NODE]
            ptq_scr[:, b * 128 + NODE:(b + 1) * 128] = (
                qt[:, b * NODE:(b + 1) * NODE])
        rr = lax.broadcasted_iota(jnp.int32, (2 * NODE, NPAIR), 0)
        pp = lax.broadcasted_iota(jnp.int32, (2 * NODE, NPAIR), 1)
        e1 = ((pp // NODE) == rr).astype(jnp.float32)
        e2 = ((pp % NODE) == (rr - NODE)).astype(jnp.float32)
        ee_scr[...] = jnp.where(rr < NODE, e1, e2).astype(jnp.bfloat16)
        w35_scr[...] = _dot3(wout8t_ref[...], wm3t_ref[...])
        b35_scr[...] = _dot3(wout8t_ref[...], bm3c_ref[...]) + bout8c_ref[...]
        out_ref[...] = jnp.zeros((1, 1), jnp.float32)

    pq = ptq_scr[:, pl.ds(pid * 128, 128)]
    h1t = jnp.maximum(_dot_sbf(pq, ee_scr[...]), 0.0)
    h2t = jnp.maximum(_dot3(wm2t_ref[...], h1t) + bm2c_ref[...], 0.0)
    logitst = _dot3(w35_scr[...], h2t) + b35_scr[...]
    mx = jnp.max(logitst, axis=0, keepdims=True)
    lse = jnp.log(jnp.sum(jnp.exp(logitst - mx), axis=0, keepdims=True)) + mx
    gold = gold_ref[...].reshape(1, NPAIR)
    onehott = (lax.broadcasted_iota(jnp.int32, (8, NPAIR), 0)
               == gold).astype(jnp.float32)
    pick = jnp.sum(logitst * onehott)
    out_ref[...] += jnp.reshape(jnp.sum(lse) - pick, (1, 1))


def _dense_forward(at_bf, st, xt, props_row, wprop_col, bprop_col, waddt,
                   badd_col, wint, bin_col, wmsgt, wedge8t, wupdt, bupd_col,
                   wm1t, bm1_col, wm2t, bm2_col, wm3t, bm3_col, wout8t,
                   bout8_col, gold3, interpret=False):
    full = lambda shape: pl.BlockSpec(shape, lambda b: (0,) * len(shape))
    out = pl.pallas_call(
        _tc_body,
        grid=(B,),
        in_specs=[
            full((N, N)), full((8, N)), full((NODE_DIM, N)),
            full((1, B)), full((B, 1)), full((B, 1)),
            full((HID, 96)), full((HID, 1)),
            full((HID, NODE_DIM)), full((HID, 1)),
            full((HID, HID)), full((HID, 8)),
            full((HID, 2 * HID)), full((HID, 1)),
            full((HID, 2 * HID)), full((HID, 1)),
            full((HID, HID)), full((HID, 1)),
            full((2 * HID, HID)), full((2 * HID, 1)),
            full((8, 2 * HID)), full((8, 1)),
            pl.BlockSpec((1, 1, NPAIR), lambda b: (b, 0, 0)),
        ],
        out_specs=pl.BlockSpec((1, 1), lambda b: (0, 0)),
        out_shape=jax.ShapeDtypeStruct((1, 1), jnp.float32),
        scratch_shapes=[
            pltpu.VMEM((HID, 2 * N), jnp.float32),
            pltpu.VMEM((HID, N), jnp.float32),
            pltpu.VMEM((HID, N), jnp.bfloat16),
            pltpu.VMEM((HID, N), jnp.bfloat16),
            pltpu.VMEM((2 * NODE, NPAIR), jnp.bfloat16),
            pltpu.VMEM((8, HID), jnp.float32),
            pltpu.VMEM((8, 1), jnp.float32),
            pltpu.VMEM((N, N), jnp.bfloat16),
        ],
        interpret=interpret,
    )(at_bf, st, xt, props_row, wprop_col, bprop_col, waddt, badd_col,
      wint, bin_col, wmsgt, wedge8t, wupdt, bupd_col, wm1t, bm1_col,
      wm2t, bm2_col, wm3t, bm3_col, wout8t, bout8_col, gold3)
    return out


@jax.jit
def kernel(x, edge_index, edge_attr, pieces, edge_select, golden_edge,
           props, W_in, b_in, W_msg, W_edge, W_upd, b_upd, W_prop, b_prop,
           W_add, b_add, W_m1, b_m1, W_m2, b_m2, W_m3, b_m3, W_out, b_out):
    src = edge_index[0].astype(jnp.int32)
    dst = edge_index[1].astype(jnp.int32)
    ea_cols = [edge_attr[:, k] for k in range(NET)]

    at_flat, st = _build_adjacency(src, dst, ea_cols)
    at_mat = at_flat.reshape(N, N)

    xt = x.reshape(N, NODE_DIM).T
    wedge8t = jnp.zeros((HID, 8), jnp.float32).at[:, 0:NET].set(W_edge.T)
    wout8t = jnp.zeros((8, 2 * HID), jnp.float32).at[0:NET, :].set(W_out.T)
    bout8c = jnp.full((8, 1), -1e30, jnp.float32).at[0:NET, 0].set(b_out)
    gold3 = golden_edge.astype(jnp.int32).reshape(B, 1, NPAIR)

    out = _dense_forward(
        at_mat, st, xt, props.reshape(1, B), W_prop.reshape(B, 1),
        b_prop.reshape(B, 1), W_add.T, b_add.reshape(HID, 1),
        W_in.T, b_in.reshape(HID, 1), W_msg.T, wedge8t, W_upd.T,
        b_upd.reshape(HID, 1), W_m1.T, b_m1.reshape(HID, 1), W_m2.T,
        b_m2.reshape(HID, 1), W_m3.T, b_m3.reshape(2 * HID, 1), wout8t,
        bout8c, gold3)
    return out[0, 0] / jnp.float32(B * NODE * NODE)


# speedup vs baseline: 20.6438x; 1.0208x over previous
import jax
import jax.numpy as jnp
from jax import lax
from jax.experimental import pallas as pl
from jax.experimental.pallas import tpu as pltpu
from jax.experimental.pallas import tpu_sc as plsc

B = 32
NODE = 64
NODE_DIM = 160
HID = 64
NET = 5
N = B * NODE
E = 32768
NPAIR = NODE * NODE


_NW = 32
_ROWS = 32
_SLAB = _ROWS * N
_CHUNK = 4096
_NCHUNK = E // _CHUNK


def _sc_body(src_hbm, dst_hbm, ea0, ea1, ea2, ea3, ea4,
             a_hbm, s_hbm, abuf, sbuf, srcb, dstb,
             eb0, eb1, eb2, eb3, eb4, sem0, sem1):
    wid = lax.axis_index("s") * 2 + lax.axis_index("c")

    zeros16 = jnp.zeros((16,), jnp.float32)
    ones16 = jnp.ones((16,), jnp.float32)
    ea_hbm = (ea0, ea1, ea2, ea3, ea4)
    eabs = (eb0, eb1, eb2, eb3, eb4)
    sems = (sem0, sem1)

    def zero_a(i, _):
        for u in range(4):
            abuf[pl.ds(i * 64 + u * 16, 16)] = zeros16
        return _

    def issue(chunk, p):
        par = chunk % 2
        base = chunk * _CHUNK
        half = pl.ds(base, _CHUNK)
        dsts = pl.ds(par * _CHUNK, _CHUNK)
        cps = [pltpu.make_async_copy(src_hbm.at[half], srcb.at[dsts],
                                     sems[par]),
               pltpu.make_async_copy(dst_hbm.at[half], dstb.at[dsts],
                                     sems[par])]
        if p == 0:
            for k in range(NET):
                cps.append(pltpu.make_async_copy(
                    ea_hbm[k].at[half], eabs[k].at[dsts], sems[par]))
        for cp in cps:
            cp.start()
        return cps

    for p in range(2):
        c = wid + p * _NW
        pend = issue(0, p)
        lax.fori_loop(0, _SLAB // 64, zero_a, None)
        if p == 0:
            def zero_s(i, _):
                sbuf[pl.ds(i * 16, 16)] = zeros16
                return _
            lax.fori_loop(0, (8 * NODE) // 16, zero_s, None)

        for chunk in range(_NCHUNK):
            par = chunk % 2
            boff = par * _CHUNK
            for cp in pend:
                cp.wait()
            if chunk + 1 < _NCHUNK:
                pend = issue(chunk + 1, p)

            def scan_vec(v, _):
                for u in range(8):
                    o = pl.ds(boff + v * 128 + u * 16, 16)
                    s = srcb[o]
                    d = dstb[o]
                    ok = (s >> 5) == c
                    row = jnp.where(ok, s - c * _ROWS, 0)
                    plsc.addupdate_scatter(
                        abuf, [row * N + d], ones16, mask=ok)
                    if p == 0:
                        oks = (d >> 6) == wid
                        dls = jnp.where(oks, d - wid * NODE, 0)
                        for k in range(NET):
                            plsc.addupdate_scatter(
                                sbuf, [k * NODE + dls], eabs[k][o], mask=oks)
                return _

            lax.fori_loop(0, _CHUNK // 128, scan_vec, None)

        pltpu.sync_copy(abuf, a_hbm.at[pl.ds(c * _ROWS * N, _SLAB)])

    for k in range(8):
        pltpu.sync_copy(sbuf.at[pl.ds(k * NODE, NODE)],
                        s_hbm.at[pl.ds(k * N + wid * NODE, NODE)])


def _build_adjacency(src, dst, ea_cols):
    mesh = plsc.VectorSubcoreMesh(core_axis_name="c", subcore_axis_name="s")
    f = pl.kernel(
        _sc_body,
        out_type=(
            jax.ShapeDtypeStruct((N * N,), jnp.float32),
            jax.ShapeDtypeStruct((8 * N,), jnp.float32),
        ),
        mesh=mesh,
        compiler_params=pltpu.CompilerParams(
            needs_layout_passes=False, use_tc_tiling_on_sc=False),
        scratch_types=[
            pltpu.VMEM((_SLAB,), jnp.float32),
            pltpu.VMEM((8 * NODE,), jnp.float32),
            pltpu.VMEM((2 * _CHUNK,), jnp.int32),
            pltpu.VMEM((2 * _CHUNK,), jnp.int32),
            pltpu.VMEM((2 * _CHUNK,), jnp.float32),
            pltpu.VMEM((2 * _CHUNK,), jnp.float32),
            pltpu.VMEM((2 * _CHUNK,), jnp.float32),
            pltpu.VMEM((2 * _CHUNK,), jnp.float32),
            pltpu.VMEM((2 * _CHUNK,), jnp.float32),
            pltpu.SemaphoreType.DMA,
            pltpu.SemaphoreType.DMA,
        ],
    )
    at_flat, st_flat = f(src, dst, ea_cols[0], ea_cols[1], ea_cols[2],
                         ea_cols[3], ea_cols[4])
    return at_flat, st_flat.reshape(8, N)


def _dot(a, b):
    return jnp.dot(a, b, precision=lax.Precision.HIGHEST,
                   preferred_element_type=jnp.float32)


def _split(a):
    hi = a.astype(jnp.bfloat16)
    lo = (a - hi.astype(jnp.float32)).astype(jnp.bfloat16)
    return hi, lo


def _dot_bf(a, b):
    return jnp.dot(a, b, preferred_element_type=jnp.float32)


def _dot_sbf(a, b_bf):
    hi, lo = _split(a)
    return _dot_bf(hi, b_bf) + _dot_bf(lo, b_bf)


def _dot3(a, b):
    ahi, alo = _split(a)
    bhi, blo = _split(b)
    return _dot_bf(ahi, bhi) + _dot_bf(alo, bhi) + _dot_bf(ahi, blo)


def _tc_body(at_ref, st_ref, xt_ref, props_ref, wpropc_ref, bpropc_ref,
             waddt_ref, baddc_ref, wint_ref, binc_ref, wmsgt_ref,
             wedge8t_ref, wupdt_ref, bupdc_ref, wm1t_ref, bm1c_ref,
             wm2t_ref, bm2c_ref, wm3t_ref, bm3c_ref, wout8t_ref,
             bout8c_ref, gold_ref, out_ref,
             ptq_scr, ht_scr, hhi_scr, hlo_scr, ee_scr, wm2s_scr, w35s_scr,
             b35_scr, at_scr):
    pid = pl.program_id(0)

    @pl.when(pid == 0)
    def _encoder():
        for kb in range(32):
            at_scr[pl.ds(kb * 64, 64), :] = (
                at_ref[pl.ds(kb * 64, 64), :].astype(jnp.bfloat16))
        ht_scr[...] = jnp.maximum(
            _dot3(wint_ref[...], xt_ref[...]) + binc_ref[...], 0.0)
        ct = _dot3(wedge8t_ref[...], st_ref[...])
        u1t = wupdt_ref[:, 0:HID]
        u2t = wupdt_ref[:, HID:2 * HID]
        wmsgt = wmsgt_ref[...]
        bupdc = bupdc_ref[...]
        for _ in range(4):
            ht = ht_scr[...]
            hhi = ht.astype(jnp.bfloat16)
            hhi_scr[...] = hhi
            hlo_scr[...] = (ht - hhi.astype(jnp.float32)).astype(jnp.bfloat16)

            def kstep(kb, acc):
                at_blk = at_scr[pl.ds(kb * 128, 128), :]
                return (acc
                        + jnp.dot(hhi_scr[:, pl.ds(kb * 128, 128)], at_blk,
                                  preferred_element_type=jnp.float32)
                        + jnp.dot(hlo_scr[:, pl.ds(kb * 128, 128)], at_blk,
                                  preferred_element_type=jnp.float32))

            gt = lax.fori_loop(0, 16, kstep,
                               jnp.zeros((HID, N), jnp.float32))
            aggt = _dot3(wmsgt, gt) + ct
            ht_scr[...] = jnp.maximum(
                _dot3(u1t, ht) + _dot3(u2t, aggt) + bupdc, 0.0)

        propt = wpropc_ref[...] * props_ref[...] + bpropc_ref[...]
        pvect = _dot3(waddt_ref[:, HID:], propt) + baddc_ref[...]
        erep2 = (lax.broadcasted_iota(jnp.int32, (B, N), 0)
                 == lax.broadcasted_iota(jnp.int32, (B, N), 1)
                 // NODE).astype(jnp.bfloat16)
        proppt = _dot_sbf(pvect, erep2)
        negt = _dot3(waddt_ref[:, 0:HID], ht_scr[...]) + proppt
        pt = _dot3(wm1t_ref[:, 0:HID], negt) + bm1c_ref[...]
        qt = _dot3(wm1t_ref[:, HID:], negt)
        for b in range(B):
            ptq_scr[:, b * 128:b * 128 + NODE] = pt[:, b * NODE:(b + 1) * NODE]
            ptq_scr[:, b * 128 + NODE:(b + 1) * 128] = (
                qt[:, b * NODE:(b + 1) * NODE])
        rr = lax.broadcasted_iota(jnp.int32, (4 * NODE, NPAIR), 0) % (2 * NODE)
        pp = lax.broadcasted_iota(jnp.int32, (4 * NODE, NPAIR), 1)
        e1 = ((pp // NODE) == rr).astype(jnp.float32)
        e2 = ((pp % NODE) == (rr - NODE)).astype(jnp.float32)
        ee_scr[...] = jnp.where(rr < NODE, e1, e2).astype(jnp.bfloat16)
        m2hi, m2lo = _split(wm2t_ref[...])
        wm2s_scr[...] = jnp.concatenate([m2hi, m2lo, m2hi], axis=1)
        w35 = _dot3(wout8t_ref[...], wm3t_ref[...])
        w5hi, w5lo = _split(w35)
        w35s_scr[...] = jnp.concatenate([w5hi, w5lo, w5hi], axis=1)
        b35_scr[...] = _dot3(wout8t_ref[...], bm3c_ref[...]) + bout8c_ref[...]
        out_ref[...] = jnp.zeros((1, 1), jnp.float32)

    pqhi, pqlo = _split(ptq_scr[:, pl.ds(pid * 128, 128)])
    pq2 = jnp.concatenate([pqhi, pqlo], axis=1)
    h1t = jnp.maximum(_dot_bf(pq2, ee_scr[...]), 0.0)
    h1hi, h1lo = _split(h1t)
    h1s = jnp.concatenate([h1hi, h1hi, h1lo], axis=0)
    h2t = jnp.maximum(_dot_bf(wm2s_scr[...], h1s) + bm2c_ref[...], 0.0)
    h2hi, h2lo = _split(h2t)
    h2s = jnp.concatenate([h2hi, h2hi, h2lo], axis=0)
    logitst = _dot_bf(w35s_scr[...], h2s) + b35_scr[...]
    mx = jnp.max(logitst, axis=0, keepdims=True)
    lse = jnp.log(jnp.sum(jnp.exp(logitst - mx), axis=0, keepdims=True)) + mx
    gold = gold_ref[...].reshape(1, NPAIR)
    onehott = (lax.broadcasted_iota(jnp.int32, (8, NPAIR), 0)
               == gold).astype(jnp.float32)
    pick = jnp.sum(logitst * onehott)
    out_ref[...] += jnp.reshape(jnp.sum(lse) - pick, (1, 1))


def _dense_forward(at_bf, st, xt, props_row, wprop_col, bprop_col, waddt,
                   badd_col, wint, bin_col, wmsgt, wedge8t, wupdt, bupd_col,
                   wm1t, bm1_col, wm2t, bm2_col, wm3t, bm3_col, wout8t,
                   bout8_col, gold3, interpret=False):
    full = lambda shape: pl.BlockSpec(shape, lambda b: (0,) * len(shape))
    out = pl.pallas_call(
        _tc_body,
        grid=(B,),
        in_specs=[
            full((N, N)), full((8, N)), full((NODE_DIM, N)),
            full((1, B)), full((B, 1)), full((B, 1)),
            full((HID, 96)), full((HID, 1)),
            full((HID, NODE_DIM)), full((HID, 1)),
            full((HID, HID)), full((HID, 8)),
            full((HID, 2 * HID)), full((HID, 1)),
            full((HID, 2 * HID)), full((HID, 1)),
            full((HID, HID)), full((HID, 1)),
            full((2 * HID, HID)), full((2 * HID, 1)),
            full((8, 2 * HID)), full((8, 1)),
            pl.BlockSpec((1, 1, NPAIR), lambda b: (b, 0, 0)),
        ],
        out_specs=pl.BlockSpec((1, 1), lambda b: (0, 0)),
        out_shape=jax.ShapeDtypeStruct((1, 1), jnp.float32),
        scratch_shapes=[
            pltpu.VMEM((HID, 2 * N), jnp.float32),
            pltpu.VMEM((HID, N), jnp.float32),
            pltpu.VMEM((HID, N), jnp.bfloat16),
            pltpu.VMEM((HID, N), jnp.bfloat16),
            pltpu.VMEM((4 * NODE, NPAIR), jnp.bfloat16),
            pltpu.VMEM((HID, 3 * HID), jnp.bfloat16),
            pltpu.VMEM((8, 3 * HID), jnp.bfloat16),
            pltpu.VMEM((8, 1), jnp.float32),
            pltpu.VMEM((N, N), jnp.bfloat16),
        ],
        interpret=interpret,
    )(at_bf, st, xt, props_row, wprop_col, bprop_col, waddt, badd_col,
      wint, bin_col, wmsgt, wedge8t, wupdt, bupd_col, wm1t, bm1_col,
      wm2t, bm2_col, wm3t, bm3_col, wout8t, bout8_col, gold3)
    return out


@jax.jit
def kernel(x, edge_index, edge_attr, pieces, edge_select, golden_edge,
           props, W_in, b_in, W_msg, W_edge, W_upd, b_upd, W_prop, b_prop,
           W_add, b_add, W_m1, b_m1, W_m2, b_m2, W_m3, b_m3, W_out, b_out):
    src = edge_index[0].astype(jnp.int32)
    dst = edge_index[1].astype(jnp.int32)
    ea_cols = [edge_attr[:, k] for k in range(NET)]

    at_flat, st = _build_adjacency(src, dst, ea_cols)
    at_mat = at_flat.reshape(N, N)

    xt = x.reshape(N, NODE_DIM).T
    wedge8t = jnp.zeros((HID, 8), jnp.float32).at[:, 0:NET].set(W_edge.T)
    wout8t = jnp.zeros((8, 2 * HID), jnp.float32).at[0:NET, :].set(W_out.T)
    bout8c = jnp.full((8, 1), -1e30, jnp.float32).at[0:NET, 0].set(b_out)
    gold3 = golden_edge.astype(jnp.int32).reshape(B, 1, NPAIR)

    out = _dense_forward(
        at_mat, st, xt, props.reshape(1, B), W_prop.reshape(B, 1),
        b_prop.reshape(B, 1), W_add.T, b_add.reshape(HID, 1),
        W_in.T, b_in.reshape(HID, 1), W_msg.T, wedge8t, W_upd.T,
        b_upd.reshape(HID, 1), W_m1.T, b_m1.reshape(HID, 1), W_m2.T,
        b_m2.reshape(HID, 1), W_m3.T, b_m3.reshape(2 * HID, 1), wout8t,
        bout8c, gold3)
    return out[0, 0] / jnp.float32(B * NODE * NODE)
